# initial kernel scaffold (unmeasured)
import jax
import jax.numpy as jnp
from jax import lax
from jax.experimental import pallas as pl
from jax.experimental.pallas import tpu as pltpu

N_DEV = 4


def kernel(x, W):
    t, d = x.shape
    _, v_sh = W.shape
    v_full = N_DEV * v_sh

    def body(x_ref, w_ref, out_ref, logits_ref, comm_ref, send_sems, recv_sems):
        my_pos = lax.axis_index("i")
        left = (my_pos - 1) % N_DEV
        right = (my_pos + 1) % N_DEV

        barrier_sem = pltpu.get_barrier_semaphore()
        for nbr in [left, right]:
            pl.semaphore_signal(
                barrier_sem, inc=1,
                device_id=(nbr,), device_id_type=pl.DeviceIdType.MESH,
            )
        pl.semaphore_wait(barrier_sem, 2)

        local = jnp.dot(
            x_ref[:, :].astype(jnp.bfloat16),
            w_ref[:, :].astype(jnp.bfloat16),
            preferred_element_type=jnp.float32,
        ).astype(jnp.bfloat16)
        logits_ref[:, pl.ds(my_pos * v_sh, v_sh)] = local
        comm_ref[0, :, :] = local

        for h in range(N_DEV - 1):
            send_slot = h % 2
            recv_slot = (h + 1) % 2
            rdma = pltpu.make_async_remote_copy(
                src_ref=comm_ref.at[send_slot],
                dst_ref=comm_ref.at[recv_slot],
                send_sem=send_sems.at[send_slot],
                recv_sem=recv_sems.at[recv_slot],
                device_id=(right,),
                device_id_type=pl.DeviceIdType.MESH,
            )
            rdma.start()
            rdma.wait()

            origin = (my_pos - h - 1) % N_DEV
            logits_ref[:, pl.ds(origin * v_sh, v_sh)] = comm_ref[recv_slot, :, :]

        logits = logits_ref[:, :].astype(jnp.float32)
        m = jnp.max(logits, axis=-1, keepdims=True)
        e = jnp.exp(logits - m)
        out_ref[:, :] = e / jnp.sum(e, axis=-1, keepdims=True)

    return pl.pallas_call(
        body,
        out_shape=jax.ShapeDtypeStruct((t, v_full), jnp.float32),
        in_specs=[
            pl.BlockSpec(memory_space=pltpu.VMEM),
            pl.BlockSpec(memory_space=pltpu.VMEM),
        ],
        out_specs=pl.BlockSpec(memory_space=pltpu.VMEM),
        scratch_shapes=[
            pltpu.VMEM((t, v_full), jnp.bfloat16),
            pltpu.VMEM((2, t, v_sh), jnp.bfloat16),
            pltpu.SemaphoreType.DMA((2,)),
            pltpu.SemaphoreType.DMA((2,)),
        ],
        compiler_params=pltpu.CompilerParams(collective_id=0),
    )(x, W)


# baseline (device time: 103684 ns/iter reference)
import jax
import jax.numpy as jnp
from jax import lax
from jax.experimental import pallas as pl
from jax.experimental.pallas import tpu as pltpu

N_DEV = 4


def kernel(x, W):
    t, d = x.shape
    _, v_sh = W.shape
    v_full = N_DEV * v_sh

    def body(x_ref, w_ref, out_ref, comm_ref, send_sems, recv_sems):
        my_pos = lax.axis_index("i")
        left = (my_pos - 1) % N_DEV
        right = (my_pos + 1) % N_DEV

        barrier_sem = pltpu.get_barrier_semaphore()
        for nbr in [left, right]:
            pl.semaphore_signal(
                barrier_sem, inc=1,
                device_id=(nbr,), device_id_type=pl.DeviceIdType.MESH,
            )
        pl.semaphore_wait(barrier_sem, 2)

        local = jnp.dot(
            x_ref[:, :].astype(jnp.bfloat16),
            w_ref[:, :].astype(jnp.bfloat16),
            preferred_element_type=jnp.float32,
        ).astype(jnp.bfloat16)
        out_ref[:, pl.ds(my_pos * v_sh, v_sh)] = local.astype(jnp.float32)
        comm_ref[0, :, :] = local

        for h in range(N_DEV - 1):
            send_slot = h % 2
            recv_slot = (h + 1) % 2
            rdma = pltpu.make_async_remote_copy(
                src_ref=comm_ref.at[send_slot],
                dst_ref=comm_ref.at[recv_slot],
                send_sem=send_sems.at[send_slot],
                recv_sem=recv_sems.at[recv_slot],
                device_id=(right,),
                device_id_type=pl.DeviceIdType.MESH,
            )
            rdma.start()
            rdma.wait()

            origin = (my_pos - h - 1) % N_DEV
            out_ref[:, pl.ds(origin * v_sh, v_sh)] = comm_ref[
                recv_slot, :, :
            ].astype(jnp.float32)

        logits = out_ref[:, :]
        m = jnp.max(logits, axis=-1, keepdims=True)
        e = jnp.exp(logits - m)
        out_ref[:, :] = e / jnp.sum(e, axis=-1, keepdims=True)

    return pl.pallas_call(
        body,
        out_shape=jax.ShapeDtypeStruct((t, v_full), jnp.float32),
        in_specs=[
            pl.BlockSpec(memory_space=pltpu.VMEM),
            pl.BlockSpec(memory_space=pltpu.VMEM),
        ],
        out_specs=pl.BlockSpec(memory_space=pltpu.VMEM),
        scratch_shapes=[
            pltpu.VMEM((2, t, v_sh), jnp.bfloat16),
            pltpu.SemaphoreType.DMA((2,)),
            pltpu.SemaphoreType.DMA((2,)),
        ],
        compiler_params=pltpu.CompilerParams(
            collective_id=0,
            vmem_limit_bytes=100 * 1024 * 1024,
        ),
    )(x, W)


# device time: 67677 ns/iter; 1.5320x vs baseline; 1.5320x over previous
import jax
import jax.numpy as jnp
from jax import lax
from jax.experimental import pallas as pl
from jax.experimental.pallas import tpu as pltpu

N_DEV = 4

LOCAL, FROM_LEFT, FROM_RIGHT, DIAG = 0, 1, 2, 3
SEND_RIGHT, SEND_LEFT, FWD_RIGHT, FWD_LEFT = 0, 1, 2, 3


def kernel(x, W):
    t, d = x.shape
    _, v_sh = W.shape
    v_full = N_DEV * v_sh
    th = t // 2

    def body(x_ref, w_ref, out_ref, comm_ref, send_sems, recv_sems):
        my_pos = lax.axis_index("i")
        left = (my_pos - 1) % N_DEV
        right = (my_pos + 1) % N_DEV
        diag = (my_pos + 2) % N_DEV

        barrier_sem = pltpu.get_barrier_semaphore()
        for nbr in [left, right]:
            pl.semaphore_signal(
                barrier_sem, inc=1,
                device_id=(nbr,), device_id_type=pl.DeviceIdType.MESH,
            )
        pl.semaphore_wait(barrier_sem, 2)

        local = jnp.dot(
            x_ref[:, :].astype(jnp.bfloat16),
            w_ref[:, :].astype(jnp.bfloat16),
            preferred_element_type=jnp.float32,
        ).astype(jnp.bfloat16)
        comm_ref[LOCAL, :, :] = local

        def copy(src_slot, dst_slot, sem_idx, target):
            return pltpu.make_async_remote_copy(
                src_ref=src_slot,
                dst_ref=dst_slot,
                send_sem=send_sems.at[sem_idx],
                recv_sem=recv_sems.at[sem_idx],
                device_id=(target,),
                device_id_type=pl.DeviceIdType.MESH,
            )

        snd_r = copy(comm_ref.at[LOCAL], comm_ref.at[FROM_LEFT], SEND_RIGHT, right)
        snd_l = copy(comm_ref.at[LOCAL], comm_ref.at[FROM_RIGHT], SEND_LEFT, left)
        snd_r.start()
        snd_l.start()

        out_ref[:, pl.ds(my_pos * v_sh, v_sh)] = local.astype(jnp.float32)

        snd_r.wait_recv()
        fwd_r = copy(
            comm_ref.at[FROM_LEFT, pl.ds(0, th)],
            comm_ref.at[DIAG, pl.ds(0, th)],
            FWD_RIGHT, right,
        )
        fwd_r.start()
        out_ref[:, pl.ds(left * v_sh, v_sh)] = comm_ref[FROM_LEFT, :, :].astype(
            jnp.float32
        )

        snd_l.wait_recv()
        fwd_l = copy(
            comm_ref.at[FROM_RIGHT, pl.ds(th, th)],
            comm_ref.at[DIAG, pl.ds(th, th)],
            FWD_LEFT, left,
        )
        fwd_l.start()
        out_ref[:, pl.ds(right * v_sh, v_sh)] = comm_ref[FROM_RIGHT, :, :].astype(
            jnp.float32
        )

        fwd_r.wait_recv()
        fwd_l.wait_recv()
        out_ref[:, pl.ds(diag * v_sh, v_sh)] = comm_ref[DIAG, :, :].astype(
            jnp.float32
        )

        snd_r.wait_send()
        snd_l.wait_send()
        fwd_r.wait_send()
        fwd_l.wait_send()

        logits = out_ref[:, :]
        m = jnp.max(logits, axis=-1, keepdims=True)
        e = jnp.exp(logits - m)
        out_ref[:, :] = e / jnp.sum(e, axis=-1, keepdims=True)

    return pl.pallas_call(
        body,
        out_shape=jax.ShapeDtypeStruct((t, v_full), jnp.float32),
        in_specs=[
            pl.BlockSpec(memory_space=pltpu.VMEM),
            pl.BlockSpec(memory_space=pltpu.VMEM),
        ],
        out_specs=pl.BlockSpec(memory_space=pltpu.VMEM),
        scratch_shapes=[
            pltpu.VMEM((4, t, v_sh), jnp.bfloat16),
            pltpu.SemaphoreType.DMA((4,)),
            pltpu.SemaphoreType.DMA((4,)),
        ],
        compiler_params=pltpu.CompilerParams(
            collective_id=0,
            vmem_limit_bytes=100 * 1024 * 1024,
        ),
    )(x, W)
